# baseline (speedup 1.0000x reference)
import jax
import jax.numpy as jnp
from jax import lax
from jax.experimental import pallas as pl
from jax.experimental.pallas import tpu as pltpu
from jax.experimental.pallas import tpu_sc as plsc

BATCH = 1024
LM1 = 49
VOCAB = 1000
VPAD = 1024
N_CLS = 100
CPAD = 112
NSRC = CPAD // 16
NC = 2
NS = 16
NW = NC * NS
VSZ = 32
HSZ = 16
NH = VSZ // HSZ
NBG = BATCH // 16
WPIPE = 8

_DNUMS = lax.GatherDimensionNumbers(offset_dims=(), collapsed_slice_dims=(0,),
                                    start_index_map=(0,))


def _sc_body(labels_hbm, wt_hbm, out_hbm, lab_v, idx_v, src_v, wtab, chunk,
             wsem, csem):
    wid = lax.axis_index("s") * NC + lax.axis_index("c")
    voff = pl.multiple_of(jnp.where(wid < NW - 1, wid * VSZ, VOCAB - VSZ), 8)

    cp1 = pltpu.async_copy(labels_hbm, lab_v, csem)
    cp2 = pltpu.async_copy(wt_hbm.at[pl.ds(voff, VSZ), :], wtab, csem)
    cp1.wait()
    cp2.wait()

    def prep(g, _):
        lab = lab_v[pl.ds(16 * g, 16)]
        idx_v[pl.ds(16 * g, 16)] = lab & 15
        src_v[pl.ds(16 * g, 16)] = lab >> 4
        return 0

    lax.fori_loop(0, NBG, prep, 0)

    def build_row(v, _):
        cols = [wtab[v, pl.ds(16 * s, 16)] for s in range(NSRC)]
        for g in range(NBG):
            idx = idx_v[pl.ds(16 * g, 16)][:, None]
            src_of = src_v[pl.ds(16 * g, 16)]
            acc = jnp.zeros((16,), jnp.float32)
            for s in range(NSRC):
                val = lax.gather(cols[s], idx, _DNUMS, (1,),
                                 mode=lax.GatherScatterMode.PROMISE_IN_BOUNDS)
                acc = jnp.where(src_of == s, val, acc)
            chunk[v, pl.ds(16 * g, 16)] = acc
        return 0

    def wait_one():
        pltpu.make_async_copy(
            chunk.at[pl.ds(0, HSZ), :],
            out_hbm.at[0, pl.ds(voff, HSZ), :], wsem).wait()

    for h in range(NH):
        lax.fori_loop(h * HSZ, (h + 1) * HSZ, build_row, 0)

        def write_plane(j, _, h=h):
            pltpu.async_copy(
                chunk.at[pl.ds(h * HSZ, HSZ), :],
                out_hbm.at[j, pl.ds(voff + h * HSZ, HSZ), :], wsem)

            @pl.when(h * LM1 + j >= WPIPE)
            def _wait():
                wait_one()

            return 0

        lax.fori_loop(0, LM1, write_plane, 0)

    for _ in range(WPIPE):
        wait_one()


@jax.jit
def _bow_broadcast(labels, w_t):
    mesh = plsc.VectorSubcoreMesh(core_axis_name="c", subcore_axis_name="s",
                                  num_cores=NC, num_subcores=NS)
    return pl.kernel(
        _sc_body,
        out_type=jax.ShapeDtypeStruct((LM1, VOCAB, BATCH), jnp.float32),
        mesh=mesh,
        scratch_types=[
            pltpu.VMEM((BATCH,), jnp.int32),
            pltpu.VMEM((BATCH,), jnp.int32),
            pltpu.VMEM((BATCH,), jnp.int32),
            pltpu.VMEM((VSZ, CPAD), jnp.float32),
            pltpu.VMEM((VSZ, BATCH), jnp.float32),
            pltpu.SemaphoreType.DMA,
            pltpu.SemaphoreType.DMA,
        ],
    )(labels, w_t)


def kernel(labels, x, W_word, W_label):
    w_t = jnp.pad(W_word.T, ((0, VPAD - VOCAB), (0, CPAD - N_CLS)))
    out_t = _bow_broadcast(labels.astype(jnp.int32), w_t)
    word_logits = jnp.transpose(out_t, (2, 0, 1))
    return (word_logits,)


# speedup vs baseline: 1.0391x; 1.0391x over previous
import jax
import jax.numpy as jnp
from jax import lax
from jax.experimental import pallas as pl
from jax.experimental.pallas import tpu as pltpu
from jax.experimental.pallas import tpu_sc as plsc

BATCH = 1024
LM1 = 49
VOCAB = 1000
VPAD = 1024
N_CLS = 100
CPAD = 112
NSRC = CPAD // 16
NC = 2
NS = 16
NW = NC * NS
VSZ = 32
HSZ = 16
NH = VSZ // HSZ
NBG = BATCH // 16
WPIPE = 24

_DNUMS = lax.GatherDimensionNumbers(offset_dims=(), collapsed_slice_dims=(0,),
                                    start_index_map=(0,))


def _sc_body(labels_hbm, wt_hbm, out_hbm, lab_v, idx_v, src_v, wtab, chunk,
             wsem, csem):
    wid = lax.axis_index("s") * NC + lax.axis_index("c")
    voff = pl.multiple_of(jnp.where(wid < NW - 1, wid * VSZ, VOCAB - VSZ), 8)

    cp1 = pltpu.async_copy(labels_hbm, lab_v, csem)
    cp2 = pltpu.async_copy(wt_hbm.at[pl.ds(voff, VSZ), :], wtab, csem)
    cp1.wait()
    cp2.wait()

    def prep(g, _):
        lab = lab_v[pl.ds(16 * g, 16)]
        idx_v[pl.ds(16 * g, 16)] = lab & 15
        src_v[pl.ds(16 * g, 16)] = lab >> 4
        return 0

    lax.fori_loop(0, NBG, prep, 0)

    def build_row(v, _):
        cols = [wtab[v, pl.ds(16 * s, 16)] for s in range(NSRC)]
        for g in range(NBG):
            idx = idx_v[pl.ds(16 * g, 16)][:, None]
            src_of = src_v[pl.ds(16 * g, 16)]
            acc = jnp.zeros((16,), jnp.float32)
            for s in range(NSRC):
                val = lax.gather(cols[s], idx, _DNUMS, (1,),
                                 mode=lax.GatherScatterMode.PROMISE_IN_BOUNDS)
                acc = jnp.where(src_of == s, val, acc)
            chunk[v, pl.ds(16 * g, 16)] = acc
        return 0

    def wait_one():
        pltpu.make_async_copy(
            chunk.at[pl.ds(0, HSZ), :],
            out_hbm.at[0, pl.ds(voff, HSZ), :], wsem).wait()

    for h in range(NH):
        lax.fori_loop(h * HSZ, (h + 1) * HSZ, build_row, 0)

        def write_plane(j, _, h=h):
            pltpu.async_copy(
                chunk.at[pl.ds(h * HSZ, HSZ), :],
                out_hbm.at[j, pl.ds(voff + h * HSZ, HSZ), :], wsem)

            @pl.when(h * LM1 + j >= WPIPE)
            def _wait():
                wait_one()

            return 0

        lax.fori_loop(0, LM1, write_plane, 0)

    for _ in range(WPIPE):
        wait_one()


@jax.jit
def _bow_broadcast(labels, w_t):
    mesh = plsc.VectorSubcoreMesh(core_axis_name="c", subcore_axis_name="s",
                                  num_cores=NC, num_subcores=NS)
    return pl.kernel(
        _sc_body,
        out_type=jax.ShapeDtypeStruct((LM1, VOCAB, BATCH), jnp.float32),
        mesh=mesh,
        scratch_types=[
            pltpu.VMEM((BATCH,), jnp.int32),
            pltpu.VMEM((BATCH,), jnp.int32),
            pltpu.VMEM((BATCH,), jnp.int32),
            pltpu.VMEM((VSZ, CPAD), jnp.float32),
            pltpu.VMEM((VSZ, BATCH), jnp.float32),
            pltpu.SemaphoreType.DMA,
            pltpu.SemaphoreType.DMA,
        ],
    )(labels, w_t)


def kernel(labels, x, W_word, W_label):
    w_t = jnp.pad(W_word.T, ((0, VPAD - VOCAB), (0, CPAD - N_CLS)))
    out_t = _bow_broadcast(labels.astype(jnp.int32), w_t)
    word_logits = jnp.transpose(out_t, (2, 0, 1))
    return (word_logits,)


# speedup vs baseline: 1.0989x; 1.0575x over previous
import jax
import jax.numpy as jnp
from jax import lax
from jax.experimental import pallas as pl
from jax.experimental.pallas import tpu as pltpu
from jax.experimental.pallas import tpu_sc as plsc

BATCH = 1024
LM1 = 49
VOCAB = 1000
VPAD = 1024
N_CLS = 100
CPAD = 112
NSRC = CPAD // 16
NC = 2
NS = 16
NW = NC * NS
VSZ = 32
BLOCKS = (8, 24)
NBG = BATCH // 16
WPIPE = 24

_DNUMS = lax.GatherDimensionNumbers(offset_dims=(), collapsed_slice_dims=(0,),
                                    start_index_map=(0,))


def _sc_body(labels_hbm, wt_hbm, out_hbm, lab_v, idx_v, src_v, wtab, chunk,
             wsem, csem):
    wid = lax.axis_index("s") * NC + lax.axis_index("c")
    voff = pl.multiple_of(jnp.where(wid < NW - 1, wid * VSZ, VOCAB - VSZ), 8)

    cp1 = pltpu.async_copy(labels_hbm, lab_v, csem)
    cp2 = pltpu.async_copy(wt_hbm.at[pl.ds(voff, VSZ), :], wtab, csem)
    cp1.wait()
    cp2.wait()

    def prep(g, _):
        lab = lab_v[pl.ds(16 * g, 16)]
        idx_v[pl.ds(16 * g, 16)] = lab & 15
        src_v[pl.ds(16 * g, 16)] = lab >> 4
        return 0

    lax.fori_loop(0, NBG, prep, 0)

    def build_pair(p, _):
        v0 = 2 * p
        cols = [[wtab[v0 + r, pl.ds(16 * s, 16)] for s in range(NSRC)]
                for r in range(2)]
        for g in range(NBG):
            idx = idx_v[pl.ds(16 * g, 16)][:, None]
            src_of = src_v[pl.ds(16 * g, 16)]
            acc = [jnp.zeros((16,), jnp.float32) for _ in range(2)]
            for s in range(NSRC):
                sel = src_of == s
                for r in range(2):
                    val = lax.gather(
                        cols[r][s], idx, _DNUMS, (1,),
                        mode=lax.GatherScatterMode.PROMISE_IN_BOUNDS)
                    acc[r] = jnp.where(sel, val, acc[r])
            for r in range(2):
                chunk[v0 + r, pl.ds(16 * g, 16)] = acc[r]
        return 0

    def wait_one(sz):
        pltpu.make_async_copy(
            chunk.at[pl.ds(0, sz), :],
            out_hbm.at[0, pl.ds(voff, sz), :], wsem).wait()

    h0, h1 = BLOCKS

    lax.fori_loop(0, h0 // 2, build_pair, 0)

    def write_plane0(j, _):
        pltpu.async_copy(chunk.at[pl.ds(0, h0), :],
                         out_hbm.at[j, pl.ds(voff, h0), :], wsem)

        @pl.when(j >= WPIPE)
        def _wait():
            wait_one(h0)

        return 0

    lax.fori_loop(0, LM1, write_plane0, 0)

    lax.fori_loop(h0 // 2, VSZ // 2, build_pair, 0)

    def write_plane1(j, _):
        pltpu.async_copy(chunk.at[pl.ds(h0, h1), :],
                         out_hbm.at[j, pl.ds(voff + h0, h1), :], wsem)

        @pl.when(j < WPIPE)
        def _wait_prev():
            wait_one(h0)

        @pl.when(j >= WPIPE)
        def _wait_own():
            wait_one(h1)

        return 0

    lax.fori_loop(0, LM1, write_plane1, 0)
    for _ in range(WPIPE):
        wait_one(h1)


@jax.jit
def _bow_broadcast(labels, w_t):
    mesh = plsc.VectorSubcoreMesh(core_axis_name="c", subcore_axis_name="s",
                                  num_cores=NC, num_subcores=NS)
    return pl.kernel(
        _sc_body,
        out_type=jax.ShapeDtypeStruct((LM1, VOCAB, BATCH), jnp.float32),
        mesh=mesh,
        scratch_types=[
            pltpu.VMEM((BATCH,), jnp.int32),
            pltpu.VMEM((BATCH,), jnp.int32),
            pltpu.VMEM((BATCH,), jnp.int32),
            pltpu.VMEM((VSZ, CPAD), jnp.float32),
            pltpu.VMEM((VSZ, BATCH), jnp.float32),
            pltpu.SemaphoreType.DMA,
            pltpu.SemaphoreType.DMA,
        ],
    )(labels, w_t)


def kernel(labels, x, W_word, W_label):
    w_t = jnp.pad(W_word.T, ((0, VPAD - VOCAB), (0, CPAD - N_CLS)))
    out_t = _bow_broadcast(labels.astype(jnp.int32), w_t)
    word_logits = jnp.transpose(out_t, (2, 0, 1))
    return (word_logits,)


# speedup vs baseline: 1.1049x; 1.0055x over previous
import jax
import jax.numpy as jnp
from jax import lax
from jax.experimental import pallas as pl
from jax.experimental.pallas import tpu as pltpu
from jax.experimental.pallas import tpu_sc as plsc

BATCH = 1024
LM1 = 49
VOCAB = 1000
VPAD = 1024
N_CLS = 100
CPAD = 112
NSRC = CPAD // 16
NC = 2
NS = 16
NW = NC * NS
VSZ = 32
BLOCKS = (8, 24)
NBG = BATCH // 16
WPIPE = 40

_DNUMS = lax.GatherDimensionNumbers(offset_dims=(), collapsed_slice_dims=(0,),
                                    start_index_map=(0,))


def _sc_body(labels_hbm, wt_hbm, out_hbm, lab_v, idx_v, src_v, wtab, chunk,
             wsem, csem):
    wid = lax.axis_index("s") * NC + lax.axis_index("c")
    voff = pl.multiple_of(jnp.where(wid < NW - 1, wid * VSZ, VOCAB - VSZ), 8)

    cp1 = pltpu.async_copy(labels_hbm, lab_v, csem)
    cp2 = pltpu.async_copy(wt_hbm.at[pl.ds(voff, VSZ), :], wtab, csem)
    cp1.wait()
    cp2.wait()

    def prep(g, _):
        lab = lab_v[pl.ds(16 * g, 16)]
        idx_v[pl.ds(16 * g, 16)] = lab & 15
        src_v[pl.ds(16 * g, 16)] = lab >> 4
        return 0

    lax.fori_loop(0, NBG, prep, 0)

    def build_pair(p, _):
        v0 = 2 * p
        cols = [[wtab[v0 + r, pl.ds(16 * s, 16)] for s in range(NSRC)]
                for r in range(2)]
        for g in range(NBG):
            idx = idx_v[pl.ds(16 * g, 16)][:, None]
            src_of = src_v[pl.ds(16 * g, 16)]
            acc = [jnp.zeros((16,), jnp.float32) for _ in range(2)]
            for s in range(NSRC):
                sel = src_of == s
                for r in range(2):
                    val = lax.gather(
                        cols[r][s], idx, _DNUMS, (1,),
                        mode=lax.GatherScatterMode.PROMISE_IN_BOUNDS)
                    acc[r] = jnp.where(sel, val, acc[r])
            for r in range(2):
                chunk[v0 + r, pl.ds(16 * g, 16)] = acc[r]
        return 0

    def wait_one(sz):
        pltpu.make_async_copy(
            chunk.at[pl.ds(0, sz), :],
            out_hbm.at[0, pl.ds(voff, sz), :], wsem).wait()

    h0, h1 = BLOCKS

    lax.fori_loop(0, h0 // 2, build_pair, 0)

    def write_plane0(j, _):
        pltpu.async_copy(chunk.at[pl.ds(0, h0), :],
                         out_hbm.at[j, pl.ds(voff, h0), :], wsem)

        @pl.when(j >= WPIPE)
        def _wait():
            wait_one(h0)

        return 0

    lax.fori_loop(0, LM1, write_plane0, 0)

    lax.fori_loop(h0 // 2, VSZ // 2, build_pair, 0)

    def write_plane1(j, _):
        pltpu.async_copy(chunk.at[pl.ds(h0, h1), :],
                         out_hbm.at[j, pl.ds(voff + h0, h1), :], wsem)

        @pl.when(j < WPIPE)
        def _wait_prev():
            wait_one(h0)

        @pl.when(j >= WPIPE)
        def _wait_own():
            wait_one(h1)

        return 0

    lax.fori_loop(0, LM1, write_plane1, 0)
    for _ in range(WPIPE):
        wait_one(h1)


@jax.jit
def _bow_broadcast(labels, w_t):
    mesh = plsc.VectorSubcoreMesh(core_axis_name="c", subcore_axis_name="s",
                                  num_cores=NC, num_subcores=NS)
    return pl.kernel(
        _sc_body,
        out_type=jax.ShapeDtypeStruct((LM1, VOCAB, BATCH), jnp.float32),
        mesh=mesh,
        scratch_types=[
            pltpu.VMEM((BATCH,), jnp.int32),
            pltpu.VMEM((BATCH,), jnp.int32),
            pltpu.VMEM((BATCH,), jnp.int32),
            pltpu.VMEM((VSZ, CPAD), jnp.float32),
            pltpu.VMEM((VSZ, BATCH), jnp.float32),
            pltpu.SemaphoreType.DMA,
            pltpu.SemaphoreType.DMA,
        ],
    )(labels, w_t)


def kernel(labels, x, W_word, W_label):
    w_t = jnp.pad(W_word.T, ((0, VPAD - VOCAB), (0, CPAD - N_CLS)))
    out_t = _bow_broadcast(labels.astype(jnp.int32), w_t)
    word_logits = jnp.transpose(out_t, (2, 0, 1))
    return (word_logits,)
